# baseline (speedup 1.0000x reference)
import functools

import jax
import jax.numpy as jnp
from jax import lax
from jax.experimental import pallas as pl
from jax.experimental.pallas import tpu as pltpu
from jax.experimental.pallas import tpu_sc as plsc

B, D_IN, H, D_OUT, A = 4096, 1024, 2048, 1024, 1000
AP = 1024
BLK = 256
SBLK = 256
NEG_INF = float("-inf")


def _mlp_body(x_ref, w1_ref, b1_ref, w2_ref, b2_ref, w3_ref, b3_ref,
              wf_ref, bf_ref, g_ref, probs_ref, logp_ref, samp_ref):
    x = x_ref[...]
    h = jnp.maximum(
        jnp.dot(x, w1_ref[...], preferred_element_type=jnp.float32,
                precision=jax.lax.Precision.DEFAULT) + b1_ref[...][None, :], 0.0)
    h = jnp.maximum(
        jnp.dot(h, w2_ref[...], preferred_element_type=jnp.float32,
                precision=jax.lax.Precision.DEFAULT) + b2_ref[...][None, :], 0.0)
    h = jnp.maximum(
        jnp.dot(h, w3_ref[...], preferred_element_type=jnp.float32,
                precision=jax.lax.Precision.DEFAULT) + b3_ref[...][None, :], 0.0)
    logits = jnp.dot(h, wf_ref[...], preferred_element_type=jnp.float32,
                     precision=jax.lax.Precision.DEFAULT) + bf_ref[...][None, :]
    m = jnp.max(logits, axis=1, keepdims=True)
    e = jnp.exp(logits - m)
    s = jnp.sum(e, axis=1, keepdims=True)
    p = e / s
    probs_ref[...] = p
    logp_ref[...] = jnp.log(p + (p == 0.0).astype(jnp.float32) * 1e-8)
    z = jnp.log(p + 1e-30) + g_ref[...]
    zm = jnp.max(z, axis=1, keepdims=True)
    iota = jax.lax.broadcasted_iota(jnp.int32, z.shape, 1)
    samp_ref[...] = jnp.min(jnp.where(z == zm, iota, AP), axis=1)


def _sort_body(keys_ref, out_ref):
    keys = keys_ref[...]
    n, cols = keys.shape
    idx = jax.lax.broadcasted_iota(jnp.int32, keys.shape, 0)
    row = jax.lax.broadcasted_iota(jnp.int32, keys.shape, 0)
    for k in range(10):
        dirbit = jax.lax.rem(jax.lax.shift_right_logical(row, k + 1), 2) == 1
        for j in range(k, -1, -1):
            s = 1 << j
            is_b = jax.lax.rem(jax.lax.shift_right_logical(row, j), 2) == 1
            kp = jnp.where(is_b, jnp.roll(keys, s, 0), jnp.roll(keys, -s, 0))
            ip = jnp.where(is_b, jnp.roll(idx, s, 0), jnp.roll(idx, -s, 0))
            self_first = (keys > kp) | ((keys == kp) & (idx < ip))
            keep = self_first ^ is_b ^ dirbit
            keys = jnp.where(keep, keys, kp)
            idx = jnp.where(keep, idx, ip)
    out_ref[...] = idx


NW = 32
ROWS_W = B // NW
RB = 16
NBLK_SC = ROWS_W // RB


def _sc_sort_body(keys_hbm, out_hbm, io_rows, ka, kb, ha, hb):
    wid = lax.axis_index("s") * 2 + lax.axis_index("c")
    lane = lax.iota(jnp.int32, 16)
    lane_row = lane * AP
    ones = jnp.ones((16,), jnp.int32)
    zeros = jnp.zeros((16,), jnp.int32)

    UN = 8
    NP = 1000

    def block(blk, _):
        row0 = wid * ROWS_W + blk * RB
        pltpu.sync_copy(keys_hbm.at[pl.ds(row0 * AP, RB * AP)], io_rows)

        def zero_a(b, _):
            for u in range(8):
                ha[pl.ds((b * 8 + u) * 16, 16)] = zeros
            return 0
        lax.fori_loop(0, AP // 8, zero_a, 0)

        def hist0(i8, _):
            i0 = i8 * UN
            raws = [plsc.load_gather(io_rows, [lane_row + (i0 + u)])
                    for u in range(UN)]
            hidxs = [lax.bitwise_and(jnp.int32(0x3FFFFFFF) - r, 1023) * 16
                     + lane for r in raws]
            for u in range(UN):
                plsc.addupdate_scatter(ha, [hidxs[u]], ones)
            return 0

        def prefix(hx, hy):
            def pre(b8, acc):
                b0 = b8 * 8
                hs = [hx[pl.ds((b0 + u) * 16, 16)] for u in range(8)]
                for u in range(8):
                    hx[pl.ds((b0 + u) * 16, 16)] = acc
                    if hy is not None:
                        hy[pl.ds((b0 + u) * 16, 16)] = zeros
                    acc = acc + hs[u]
                return acc
            lax.fori_loop(0, AP // 8, pre, zeros)

        def pass1(_i8, _):
            i0 = _i8 * UN
            raws = [plsc.load_gather(io_rows, [lane_row + (i0 + u)])
                    for u in range(UN)]
            ikeys = [jnp.int32(0x3FFFFFFF) - r for r in raws]
            hidxs = [lax.bitwise_and(k, 1023) * 16 + lane for k in ikeys]
            kvs = [lax.shift_left(lax.shift_right_logical(ikeys[u], 10), 10)
                   + (i0 + u) for u in range(UN)]
            h2s = [lax.bitwise_and(lax.shift_right_logical(k, 10), 1023) * 16
                   + lane for k in ikeys]
            offs = []
            for u in range(UN):
                offs.append(plsc.load_gather(ha, [hidxs[u]]))
                plsc.addupdate_scatter(ha, [hidxs[u]], ones)
            for u in range(UN):
                plsc.store_scatter(kb, [offs[u] * 16 + lane], kvs[u])
                plsc.addupdate_scatter(hb, [h2s[u]], ones)
            return 0

        def pass2(_i8, _):
            i0 = _i8 * UN
            kvs = [kb[pl.ds((i0 + u) * 16, 16)] for u in range(UN)]
            hidxs = [lax.bitwise_and(lax.shift_right_logical(kv, 10), 1023)
                     * 16 + lane for kv in kvs]
            kv2s = [lax.shift_left(lax.shift_right_logical(kv, 20), 10)
                    + lax.bitwise_and(kv, 1023) for kv in kvs]
            h2s = [lax.shift_right_logical(kv, 20) * 16 + lane for kv in kvs]
            offs = []
            for u in range(UN):
                offs.append(plsc.load_gather(hb, [hidxs[u]]))
                plsc.addupdate_scatter(hb, [hidxs[u]], ones)
            for u in range(UN):
                plsc.store_scatter(ka, [offs[u] * 16 + lane], kv2s[u])
                plsc.addupdate_scatter(ha, [h2s[u]], ones)
            return 0

        def pass3(_i8, _):
            i0 = _i8 * UN
            kvs = [ka[pl.ds((i0 + u) * 16, 16)] for u in range(UN)]
            hidxs = [lax.shift_right_logical(kv, 10) * 16 + lane
                     for kv in kvs]
            vs = [lax.bitwise_and(kv, 1023) for kv in kvs]
            offs = []
            for u in range(UN):
                offs.append(plsc.load_gather(ha, [hidxs[u]]))
                plsc.addupdate_scatter(ha, [hidxs[u]], ones)
            for u in range(UN):
                plsc.store_scatter(io_rows, [lane_row + offs[u]], vs[u])
            return 0

        lax.fori_loop(0, NP // UN, hist0, 0)
        prefix(ha, hb)
        lax.fori_loop(0, NP // UN, pass1, 0)
        prefix(hb, ha)
        lax.fori_loop(0, NP // UN, pass2, 0)
        prefix(ha, None)
        lax.fori_loop(0, NP // UN, pass3, 0)

        pltpu.sync_copy(io_rows, out_hbm.at[pl.ds(row0 * AP, RB * AP)])
        return 0

    lax.fori_loop(0, NBLK_SC, block, 0)


@functools.lru_cache(maxsize=1)
def _get_sc_sort():
    return pl.kernel(
        _sc_sort_body,
        mesh=plsc.VectorSubcoreMesh(core_axis_name="c", subcore_axis_name="s"),
        out_type=jax.ShapeDtypeStruct((B * AP,), jnp.int32),
        compiler_params=pltpu.CompilerParams(needs_layout_passes=False),
        scratch_types=[
            pltpu.VMEM((RB * AP,), jnp.int32),
            pltpu.VMEM((AP * RB,), jnp.int32),
            pltpu.VMEM((AP * RB,), jnp.int32),
            pltpu.VMEM((AP * RB,), jnp.int32),
            pltpu.VMEM((AP * RB,), jnp.int32),
        ],
    )


@functools.partial(jax.jit, static_argnames=())
def _run(state, W1, b1, W2, b2, W3, b3, Wfp, bfp, g):
    nblk = B // BLK
    probs, logp, samp = pl.pallas_call(
        _mlp_body,
        grid=(nblk,),
        in_specs=[
            pl.BlockSpec((BLK, D_IN), lambda i: (i, 0)),
            pl.BlockSpec((D_IN, H), lambda i: (0, 0)),
            pl.BlockSpec((H,), lambda i: (0,)),
            pl.BlockSpec((H, H), lambda i: (0, 0)),
            pl.BlockSpec((H,), lambda i: (0,)),
            pl.BlockSpec((H, D_OUT), lambda i: (0, 0)),
            pl.BlockSpec((D_OUT,), lambda i: (0,)),
            pl.BlockSpec((D_OUT, AP), lambda i: (0, 0)),
            pl.BlockSpec((AP,), lambda i: (0,)),
            pl.BlockSpec((BLK, AP), lambda i: (i, 0)),
        ],
        out_specs=[
            pl.BlockSpec((BLK, AP), lambda i: (i, 0)),
            pl.BlockSpec((BLK, AP), lambda i: (i, 0)),
            pl.BlockSpec((BLK,), lambda i: (i,)),
        ],
        out_shape=[
            jax.ShapeDtypeStruct((B, AP), jnp.float32),
            jax.ShapeDtypeStruct((B, AP), jnp.float32),
            jax.ShapeDtypeStruct((B,), jnp.int32),
        ],
    )(state, W1, b1, W2, b2, W3, b3, Wfp, bfp, g)

    keys_flat = jax.lax.bitcast_convert_type(probs, jnp.int32).reshape(B * AP)
    det = _get_sc_sort()(keys_flat).reshape(B, AP)
    return probs, logp, samp, det


def kernel(state, W1, b1, W2, b2, W3, b3, Wf, bf):
    Wfp = jnp.pad(Wf, ((0, 0), (0, AP - A)))
    bfp = jnp.concatenate([bf, jnp.full((AP - A,), NEG_INF, jnp.float32)])
    g = jax.random.gumbel(jax.random.key(42), (B, A), jnp.float32)
    g = jnp.concatenate([g, jnp.full((B, AP - A), NEG_INF, jnp.float32)], axis=1)
    probs, logp, samp, det = _run(state, W1, b1, W2, b2, W3, b3, Wfp, bfp, g)
    return (samp[:, None],
            (probs[:, :A], logp[:, :A]),
            det[:, :A])


# speedup vs baseline: 1.0803x; 1.0803x over previous
import functools

import jax
import jax.numpy as jnp
from jax import lax
from jax.experimental import pallas as pl
from jax.experimental.pallas import tpu as pltpu
from jax.experimental.pallas import tpu_sc as plsc

B, D_IN, H, D_OUT, A = 4096, 1024, 2048, 1024, 1000
AP = 1024
BLK = 256
SBLK = 256
NEG_INF = float("-inf")


def _mlp_body(x_ref, w1_ref, b1_ref, w2_ref, b2_ref, w3_ref, b3_ref,
              wf_ref, bf_ref, g_ref, probs_ref, logp_ref, samp_ref):
    x = x_ref[...]
    h = jnp.maximum(
        jnp.dot(x, w1_ref[...], preferred_element_type=jnp.float32,
                precision=jax.lax.Precision.DEFAULT) + b1_ref[...][None, :], 0.0)
    h = jnp.maximum(
        jnp.dot(h, w2_ref[...], preferred_element_type=jnp.float32,
                precision=jax.lax.Precision.DEFAULT) + b2_ref[...][None, :], 0.0)
    h = jnp.maximum(
        jnp.dot(h, w3_ref[...], preferred_element_type=jnp.float32,
                precision=jax.lax.Precision.DEFAULT) + b3_ref[...][None, :], 0.0)
    logits = jnp.dot(h, wf_ref[...], preferred_element_type=jnp.float32,
                     precision=jax.lax.Precision.DEFAULT) + bf_ref[...][None, :]
    m = jnp.max(logits, axis=1, keepdims=True)
    e = jnp.exp(logits - m)
    s = jnp.sum(e, axis=1, keepdims=True)
    p = e / s
    probs_ref[...] = p
    logp_ref[...] = jnp.log(p + (p == 0.0).astype(jnp.float32) * 1e-8)
    z = jnp.log(p + 1e-30) + g_ref[...]
    zm = jnp.max(z, axis=1, keepdims=True)
    iota = jax.lax.broadcasted_iota(jnp.int32, z.shape, 1)
    samp_ref[...] = jnp.min(jnp.where(z == zm, iota, AP), axis=1)


def _sort_body(keys_ref, out_ref):
    keys = keys_ref[...]
    n, cols = keys.shape
    idx = jax.lax.broadcasted_iota(jnp.int32, keys.shape, 0)
    row = jax.lax.broadcasted_iota(jnp.int32, keys.shape, 0)
    for k in range(10):
        dirbit = jax.lax.rem(jax.lax.shift_right_logical(row, k + 1), 2) == 1
        for j in range(k, -1, -1):
            s = 1 << j
            is_b = jax.lax.rem(jax.lax.shift_right_logical(row, j), 2) == 1
            kp = jnp.where(is_b, jnp.roll(keys, s, 0), jnp.roll(keys, -s, 0))
            ip = jnp.where(is_b, jnp.roll(idx, s, 0), jnp.roll(idx, -s, 0))
            self_first = (keys > kp) | ((keys == kp) & (idx < ip))
            keep = self_first ^ is_b ^ dirbit
            keys = jnp.where(keep, keys, kp)
            idx = jnp.where(keep, idx, ip)
    out_ref[...] = idx


NW = 32
ROWS_W = B // NW
RB = 16
NBLK_SC = ROWS_W // RB


def _sc_sort_body(keys_hbm, out_hbm, io_rows, ka, kb, vb, ha, hb):
    wid = lax.axis_index("s") * 2 + lax.axis_index("c")
    lane = lax.iota(jnp.int32, 16)
    lane_row = lane * AP
    ones = jnp.ones((16,), jnp.int32)
    zeros = jnp.zeros((16,), jnp.int32)

    UN = 8
    NP = 1000

    def block(blk, _):
        row0 = wid * ROWS_W + blk * RB
        pltpu.sync_copy(keys_hbm.at[pl.ds(row0 * AP, RB * AP)], io_rows)

        def zero_a(b, _):
            for u in range(8):
                ha[pl.ds((b * 8 + u) * 16, 16)] = zeros
            return 0
        lax.fori_loop(0, AP // 8, zero_a, 0)

        def ph0(i8, _):
            i0 = i8 * UN
            raws = [plsc.load_gather(io_rows, [lane_row + (i0 + u)])
                    for u in range(UN)]
            ikeys = [jnp.int32(0x3FFFFFFF) - r for r in raws]
            hidxs = [lax.bitwise_and(k, 1023) * 16 + lane for k in ikeys]
            for u in range(UN):
                ka[pl.ds((i0 + u) * 16, 16)] = ikeys[u]
            for u in range(UN):
                plsc.addupdate_scatter(ha, [hidxs[u]], ones)
            return 0

        def prefix(hx, hy):
            def pre(b8, acc):
                b0 = b8 * 8
                hs = [hx[pl.ds((b0 + u) * 16, 16)] for u in range(8)]
                for u in range(8):
                    hx[pl.ds((b0 + u) * 16, 16)] = acc
                    if hy is not None:
                        hy[pl.ds((b0 + u) * 16, 16)] = zeros
                    acc = acc + hs[u]
                return acc
            lax.fori_loop(0, AP // 8, pre, zeros)

        def permute(src_k, src_v, dst_k, dst_v, hx, hy, shift):
            def body(i8, _):
                i0 = i8 * UN
                ks = [src_k[pl.ds((i0 + u) * 16, 16)] for u in range(UN)]
                if src_v is None:
                    vs = [zeros + (i0 + u) for u in range(UN)]
                else:
                    vs = [src_v[pl.ds((i0 + u) * 16, 16)] for u in range(UN)]
                hidxs = [
                    lax.bitwise_and(lax.shift_right_logical(k, shift), 1023)
                    * 16 + lane for k in ks]
                offs = []
                for u in range(UN):
                    offs.append(plsc.load_gather(hx, [hidxs[u]]))
                    plsc.addupdate_scatter(hx, [hidxs[u]], ones)
                if dst_k is None:
                    for u in range(UN):
                        plsc.store_scatter(dst_v, [lane_row + offs[u]], vs[u])
                else:
                    didxs = [off * 16 + lane for off in offs]
                    h2s = [lax.bitwise_and(
                        lax.shift_right_logical(k, shift + 10), 1023) * 16
                        + lane for k in ks]
                    for u in range(UN):
                        plsc.store_scatter(dst_k, [didxs[u]], ks[u])
                        plsc.store_scatter(dst_v, [didxs[u]], vs[u])
                        plsc.addupdate_scatter(hy, [h2s[u]], ones)
                return 0
            lax.fori_loop(0, NP // UN, body, 0)

        lax.fori_loop(0, NP // UN, ph0, 0)
        prefix(ha, hb)
        permute(ka, None, kb, vb, ha, hb, 0)
        prefix(hb, ha)
        permute(kb, vb, ka, io_rows, hb, ha, 10)
        prefix(ha, None)
        permute(ka, io_rows, None, kb, ha, None, 20)

        pltpu.sync_copy(kb, out_hbm.at[pl.ds(row0 * AP, RB * AP)])
        return 0

    lax.fori_loop(0, NBLK_SC, block, 0)


@functools.lru_cache(maxsize=1)
def _get_sc_sort():
    return pl.kernel(
        _sc_sort_body,
        mesh=plsc.VectorSubcoreMesh(core_axis_name="c", subcore_axis_name="s"),
        out_type=jax.ShapeDtypeStruct((B * AP,), jnp.int32),
        compiler_params=pltpu.CompilerParams(needs_layout_passes=False),
        scratch_types=[
            pltpu.VMEM((RB * AP,), jnp.int32),
            pltpu.VMEM((AP * RB,), jnp.int32),
            pltpu.VMEM((AP * RB,), jnp.int32),
            pltpu.VMEM((AP * RB,), jnp.int32),
            pltpu.VMEM((AP * RB,), jnp.int32),
            pltpu.VMEM((AP * RB,), jnp.int32),
        ],
    )


@functools.partial(jax.jit, static_argnames=())
def _run(state, W1, b1, W2, b2, W3, b3, Wfp, bfp, g):
    nblk = B // BLK
    probs, logp, samp = pl.pallas_call(
        _mlp_body,
        grid=(nblk,),
        in_specs=[
            pl.BlockSpec((BLK, D_IN), lambda i: (i, 0)),
            pl.BlockSpec((D_IN, H), lambda i: (0, 0)),
            pl.BlockSpec((H,), lambda i: (0,)),
            pl.BlockSpec((H, H), lambda i: (0, 0)),
            pl.BlockSpec((H,), lambda i: (0,)),
            pl.BlockSpec((H, D_OUT), lambda i: (0, 0)),
            pl.BlockSpec((D_OUT,), lambda i: (0,)),
            pl.BlockSpec((D_OUT, AP), lambda i: (0, 0)),
            pl.BlockSpec((AP,), lambda i: (0,)),
            pl.BlockSpec((BLK, AP), lambda i: (i, 0)),
        ],
        out_specs=[
            pl.BlockSpec((BLK, AP), lambda i: (i, 0)),
            pl.BlockSpec((BLK, AP), lambda i: (i, 0)),
            pl.BlockSpec((BLK,), lambda i: (i,)),
        ],
        out_shape=[
            jax.ShapeDtypeStruct((B, AP), jnp.float32),
            jax.ShapeDtypeStruct((B, AP), jnp.float32),
            jax.ShapeDtypeStruct((B,), jnp.int32),
        ],
    )(state, W1, b1, W2, b2, W3, b3, Wfp, bfp, g)

    keys_flat = jax.lax.bitcast_convert_type(probs, jnp.int32).reshape(B * AP)
    det = _get_sc_sort()(keys_flat).reshape(B, AP)
    return probs, logp, samp, det


def kernel(state, W1, b1, W2, b2, W3, b3, Wf, bf):
    Wfp = jnp.pad(Wf, ((0, 0), (0, AP - A)))
    bfp = jnp.concatenate([bf, jnp.full((AP - A,), NEG_INF, jnp.float32)])
    g = jax.random.gumbel(jax.random.key(42), (B, A), jnp.float32)
    g = jnp.concatenate([g, jnp.full((B, AP - A), NEG_INF, jnp.float32)], axis=1)
    probs, logp, samp, det = _run(state, W1, b1, W2, b2, W3, b3, Wfp, bfp, g)
    return (samp[:, None],
            (probs[:, :A], logp[:, :A]),
            det[:, :A])
